# baseline (speedup 1.0000x reference)
import functools

import jax
import jax.numpy as jnp
from jax import lax
from jax.experimental import pallas as pl
from jax.experimental.pallas import tpu as pltpu
from jax.experimental.pallas import tpu_sc as plsc

_B, _L, _D = 512, 512, 256
_G = 16
_NC, _NS = 2, 16
_S = 320
_RPW = _S // (_NC * _NS)
_CH = 128
_NCH = _L // _CH


def _mk_tc_body(row0, with_steps):
    def body(*refs):
        mask_sref = refs[0]
        if with_steps:
            (src_hbm, st_ref, mk_ref, m3_ref, ini_ref,
             out_ref, so_ref, s0, s1, sems) = refs[1:]
        else:
            (src_hbm, m3_ref, ini_ref, out_ref, s0, s1, sems) = refs[1:]

        b = pl.program_id(0)
        n = pl.num_programs(0)

        def issue(step, buf, sem_slot):
            base = row0 + step * _G
            for j in range(_G):
                @pl.when(mask_sref[base + j] == 0)
                def _(j=j):
                    pltpu.make_async_copy(
                        src_hbm.at[pl.ds(base + j, 1)],
                        buf.at[pl.ds(j, 1)], sems.at[sem_slot, j]).start()

        def wait(step, buf, sem_slot):
            base = row0 + step * _G
            for j in range(_G):
                @pl.when(mask_sref[base + j] == 0)
                def _(j=j):
                    pltpu.make_async_copy(
                        src_hbm.at[pl.ds(base + j, 1)],
                        buf.at[pl.ds(j, 1)], sems.at[sem_slot, j]).wait()

        @pl.when(b == 0)
        def _():
            if with_steps:
                so_ref[...] = jnp.where(mk_ref[...] != 0,
                                        jnp.zeros_like(st_ref[...]),
                                        st_ref[...])
            issue(0, s0, 0)

        ini_row = jnp.broadcast_to(ini_ref[...].reshape(1, 1, _D),
                                   (_G, _L, _D))
        for p in (0, 1):
            @pl.when(lax.rem(b, 2) == p)
            def _(p=p):
                sp = s0 if p == 0 else s1
                sq = s1 if p == 0 else s0

                @pl.when(b + 1 < n)
                def _():
                    issue(b + 1, sq, 1 - p)

                wait(b, sp, p)
                out_ref[...] = jnp.where(m3_ref[...] != 0, ini_row, sp[...])

    return body


def _tc_call(y, steps, mask_i32, y_init):
    B, L, D = y.shape
    steps2d = steps.reshape(1, B)
    mask2d = mask_i32.reshape(1, B)
    mask3d = mask_i32.reshape(B, 1, 1)
    yi2d = y_init.reshape(1, D)

    grid_spec = pltpu.PrefetchScalarGridSpec(
        num_scalar_prefetch=1,
        grid=(B // _G,),
        in_specs=[
            pl.BlockSpec(memory_space=pltpu.MemorySpace.HBM),
            pl.BlockSpec((1, B), lambda i, mref: (0, 0)),
            pl.BlockSpec((1, B), lambda i, mref: (0, 0)),
            pl.BlockSpec((_G, 1, 1), lambda i, mref: (i, 0, 0)),
            pl.BlockSpec((1, D), lambda i, mref: (0, 0)),
        ],
        out_specs=[
            pl.BlockSpec((_G, L, D), lambda i, mref: (i, 0, 0)),
            pl.BlockSpec((1, B), lambda i, mref: (0, 0)),
        ],
        scratch_shapes=[
            pltpu.VMEM((_G, L, D), jnp.float32),
            pltpu.VMEM((_G, L, D), jnp.float32),
            pltpu.SemaphoreType.DMA((2, _G)),
        ],
    )

    y_new, so = pl.pallas_call(
        _mk_tc_body(0, True),
        grid_spec=grid_spec,
        out_shape=[
            jax.ShapeDtypeStruct((B, L, D), y.dtype),
            jax.ShapeDtypeStruct((1, B), steps.dtype),
        ],
        compiler_params=pltpu.CompilerParams(
            dimension_semantics=("arbitrary",),
        ),
    )(mask_i32, y, steps2d, mask2d, mask3d, yi2d)
    return y_new, so.reshape(B)


def _sc_body(z_hbm, mask_hbm, zi_hbm, zo_hbm, mvec, shbuf, shinit,
             sem_r0, sem_r1, sem_w0, sem_w1):
    cid = lax.axis_index("c")
    sid = lax.axis_index("s")
    wid = sid * _NC + cid
    base = wid * _RPW

    pltpu.sync_copy(mask_hbm.at[wid], mvec)
    pltpu.sync_copy(zi_hbm, shinit.at[sid])
    m = mvec[...]
    sem_r = (sem_r0, sem_r1)
    sem_w = (sem_w0, sem_w1)

    t = 0
    for j in range(_RPW):
        row = base + j
        s = m[j]
        for k in range(_NCH):
            slot = t % 2
            dst = zo_hbm.at[pl.ds(row, 1), pl.ds(k * _CH, _CH)]
            buf = shbuf.at[sid, slot]
            if t >= 2:
                pltpu.make_async_copy(shinit.at[sid], dst, sem_w[slot]).wait()

            @pl.when(s == 0)
            def _(dst=dst, slot=slot, row=row, k=k, buf=buf):
                src_slice = z_hbm.at[pl.ds(row, 1), pl.ds(k * _CH, _CH)]
                pltpu.async_copy(src_slice, buf, sem_r[slot])
                pltpu.make_async_copy(src_slice, buf, sem_r[slot]).wait()
                pltpu.async_copy(buf, dst, sem_w[slot])

            @pl.when(s != 0)
            def _(dst=dst, slot=slot):
                pltpu.async_copy(shinit.at[sid], dst, sem_w[slot])

            t += 1

    for slot in (0, 1):
        pltpu.make_async_copy(
            shinit.at[sid], zo_hbm.at[pl.ds(base, 1), pl.ds(0, _CH)],
            sem_w[slot]).wait()


def _sc_call(z, mask_i32, z_init):
    B, L, D = z.shape
    zi_chunk = jnp.broadcast_to(z_init, (1, _CH, D))
    mask_tab = jnp.zeros((_NC * _NS, 16), jnp.int32)
    mask_tab = mask_tab.at[:, :_RPW].set(mask_i32[:_S].reshape(_NC * _NS, _RPW))

    kfn = functools.partial(
        pl.kernel,
        mesh=plsc.VectorSubcoreMesh(core_axis_name="c", subcore_axis_name="s"),
        out_type=jax.ShapeDtypeStruct((B, L, D), z.dtype),
        scratch_types=[
            pltpu.VMEM((16,), jnp.int32),
            pltpu.VMEM_SHARED((_NS, 2, 1, _CH, _D), jnp.float32),
            pltpu.VMEM_SHARED((_NS, 1, _CH, _D), jnp.float32),
            pltpu.SemaphoreType.DMA,
            pltpu.SemaphoreType.DMA,
            pltpu.SemaphoreType.DMA,
            pltpu.SemaphoreType.DMA,
        ],
    )(_sc_body)
    return kfn(z, mask_tab, zi_chunk)


def _tc_tail_call(z, z_partial, mask_i32, z_init):
    B, L, D = z.shape
    mask3d = mask_i32.reshape(B, 1, 1)
    zi2d = z_init.reshape(1, D)

    grid_spec = pltpu.PrefetchScalarGridSpec(
        num_scalar_prefetch=1,
        grid=((B - _S) // _G,),
        in_specs=[
            pl.BlockSpec(memory_space=pltpu.MemorySpace.HBM),
            pl.BlockSpec(memory_space=pltpu.MemorySpace.HBM),
            pl.BlockSpec((_G, 1, 1), lambda i, mref: (_S // _G + i, 0, 0)),
            pl.BlockSpec((1, D), lambda i, mref: (0, 0)),
        ],
        out_specs=[
            pl.BlockSpec((_G, L, D), lambda i, mref: (_S // _G + i, 0, 0)),
        ],
        scratch_shapes=[
            pltpu.VMEM((_G, L, D), jnp.float32),
            pltpu.VMEM((_G, L, D), jnp.float32),
            pltpu.SemaphoreType.DMA((2, _G)),
        ],
    )

    def tail_body(mask_sref, z_hbm, zp_hbm, m3_ref, ini_ref, out_ref,
                  s0, s1, sems):
        inner = _mk_tc_body(_S, False)
        inner(mask_sref, z_hbm, m3_ref, ini_ref, out_ref, s0, s1, sems)

    (z_new,) = pl.pallas_call(
        tail_body,
        grid_spec=grid_spec,
        out_shape=[jax.ShapeDtypeStruct((B, L, D), z.dtype)],
        input_output_aliases={2: 0},
        compiler_params=pltpu.CompilerParams(
            dimension_semantics=("arbitrary",),
        ),
    )(mask_i32, z, z_partial, mask3d, zi2d)
    return z_new


def kernel(y, z, steps, mask, y_init, z_init):
    mask_i32 = mask.astype(jnp.int32)
    y_new, steps_new = _tc_call(y, steps, mask_i32, y_init)
    z_partial = _sc_call(z, mask_i32, z_init)
    z_new = _tc_tail_call(z, z_partial, mask_i32, z_init)
    return (y_new, z_new, steps_new)


# speedup vs baseline: 1.0506x; 1.0506x over previous
import functools

import jax
import jax.numpy as jnp
from jax import lax
from jax.experimental import pallas as pl
from jax.experimental.pallas import tpu as pltpu
from jax.experimental.pallas import tpu_sc as plsc

_B, _L, _D = 512, 512, 256
_G = 16
_NC, _NS = 2, 16
_VPW = _B // (_NC * _NS)


def _tc_body(mask_sref, y_hbm, z_hbm, yi_ref, zi_ref, yo_ref, zo_ref, sems):
    b = pl.program_id(0)
    base = b * _G

    for j in range(_G):
        @pl.when(mask_sref[base + j] == 0)
        def _(j=j):
            pltpu.make_async_copy(y_hbm.at[pl.ds(base + j, 1)],
                                  yo_ref.at[pl.ds(j, 1)], sems.at[0, j]).start()
            pltpu.make_async_copy(z_hbm.at[pl.ds(base + j, 1)],
                                  zo_ref.at[pl.ds(j, 1)], sems.at[1, j]).start()

    yi_row = jnp.broadcast_to(yi_ref[...].reshape(1, 1, _D), (1, _L, _D))
    zi_row = jnp.broadcast_to(zi_ref[...].reshape(1, 1, _D), (1, _L, _D))
    for j in range(_G):
        @pl.when(mask_sref[base + j] != 0)
        def _(j=j):
            yo_ref[pl.ds(j, 1)] = yi_row
            zo_ref[pl.ds(j, 1)] = zi_row

    for j in range(_G):
        @pl.when(mask_sref[base + j] == 0)
        def _(j=j):
            pltpu.make_async_copy(y_hbm.at[pl.ds(base + j, 1)],
                                  yo_ref.at[pl.ds(j, 1)], sems.at[0, j]).wait()
            pltpu.make_async_copy(z_hbm.at[pl.ds(base + j, 1)],
                                  zo_ref.at[pl.ds(j, 1)], sems.at[1, j]).wait()


def _tc_call(y, z, mask_i32, y_init, z_init):
    B, L, D = y.shape
    yi2d = y_init.reshape(1, D)
    zi2d = z_init.reshape(1, D)

    grid_spec = pltpu.PrefetchScalarGridSpec(
        num_scalar_prefetch=1,
        grid=(B // _G,),
        in_specs=[
            pl.BlockSpec(memory_space=pltpu.MemorySpace.HBM),
            pl.BlockSpec(memory_space=pltpu.MemorySpace.HBM),
            pl.BlockSpec((1, D), lambda i, mref: (0, 0)),
            pl.BlockSpec((1, D), lambda i, mref: (0, 0)),
        ],
        out_specs=[
            pl.BlockSpec((_G, L, D), lambda i, mref: (i, 0, 0)),
            pl.BlockSpec((_G, L, D), lambda i, mref: (i, 0, 0)),
        ],
        scratch_shapes=[
            pltpu.SemaphoreType.DMA((2, _G)),
        ],
    )

    y_new, z_new = pl.pallas_call(
        _tc_body,
        grid_spec=grid_spec,
        out_shape=[
            jax.ShapeDtypeStruct((B, L, D), y.dtype),
            jax.ShapeDtypeStruct((B, L, D), z.dtype),
        ],
        compiler_params=pltpu.CompilerParams(
            dimension_semantics=("arbitrary",),
        ),
    )(mask_i32, y, z, yi2d, zi2d)
    return y_new, z_new


def _sc_steps_body(st_hbm, mask_hbm, so_hbm, st_v, m_v, o_v, sem):
    cid = lax.axis_index("c")
    sid = lax.axis_index("s")
    wid = sid * _NC + cid
    base = wid * _VPW

    pltpu.sync_copy(st_hbm.at[pl.ds(base, _VPW)], st_v)
    pltpu.sync_copy(mask_hbm.at[pl.ds(base, _VPW)], m_v)
    o_v[...] = jnp.where(m_v[...] != 0, jnp.zeros_like(st_v[...]), st_v[...])
    pltpu.sync_copy(o_v, so_hbm.at[pl.ds(base, _VPW)])


def _sc_steps_call(steps, mask_i32):
    kfn = functools.partial(
        pl.kernel,
        mesh=plsc.VectorSubcoreMesh(core_axis_name="c", subcore_axis_name="s"),
        out_type=jax.ShapeDtypeStruct((_B,), steps.dtype),
        scratch_types=[
            pltpu.VMEM((_VPW,), jnp.int32),
            pltpu.VMEM((_VPW,), jnp.int32),
            pltpu.VMEM((_VPW,), jnp.int32),
            pltpu.SemaphoreType.DMA,
        ],
    )(_sc_steps_body)
    return kfn(steps, mask_i32)


def kernel(y, z, steps, mask, y_init, z_init):
    mask_i32 = mask.astype(jnp.int32)
    y_new, z_new = _tc_call(y, z, mask_i32, y_init, z_init)
    steps_new = _sc_steps_call(steps, mask_i32)
    return (y_new, z_new, steps_new)
